# baseline (speedup 1.0000x reference)
import functools

import jax
import jax.numpy as jnp
from jax import lax
from jax.experimental import pallas as pl
from jax.experimental.pallas import tpu as pltpu
from jax.experimental.pallas import tpu_sc as plsc

NC, NS, L = 2, 16, 16
NW = NC * NS
D = 128
SUB = 128
GROUPS = SUB // L
NBUF = 2

A_SUB = 140
B_SUB = 20
HALF_MAX = max(A_SUB, B_SUB) // 2


def _sc_body(src_hbm, dst_hbm, et_hbm, z_hbm, w_hbm, out_hbm,
             sidx, didx, eidx,
             srows0, drows0, wrows0, srows1, drows1, wrows1,
             outv0, outv1, gsem0, gsem1, osem0, osem1):
    c = lax.axis_index("c")
    s = lax.axis_index("s")
    lane = lax.iota(jnp.int32, L)
    base_sub = jnp.where(c == 0, s * A_SUB, NS * A_SUB + s * B_SUB)
    my_n = jnp.where(c == 0, A_SUB, B_SUB)
    tbase = base_sub * SUB
    row0 = base_sub

    half = my_n // 2
    rows = ((srows0, drows0, wrows0), (srows1, drows1, wrows1))
    outvs = (outv0, outv1)
    gsems = (gsem0, gsem1)
    osems = (osem0, osem1)

    def fire(j, b):
        s, d, w = rows[b]
        pltpu.async_copy(z_hbm.at[sidx.at[j]], s, gsems[b])
        pltpu.async_copy(z_hbm.at[didx.at[j]], d, gsems[b])
        pltpu.async_copy(w_hbm.at[eidx.at[j]], w, gsems[b])

    def wait_rows(b):
        s, d, w = rows[b]
        pltpu.make_async_copy(z_hbm.at[sidx.at[0]], s, gsems[b]).wait()
        pltpu.make_async_copy(z_hbm.at[didx.at[0]], d, gsems[b]).wait()
        pltpu.make_async_copy(w_hbm.at[eidx.at[0]], w, gsems[b]).wait()

    def wait_out(b):
        pltpu.make_async_copy(outvs[b], out_hbm.at[pl.ds(tbase, SUB)],
                              osems[b]).wait()

    def compute(b):
        srows_b, drows_b, wrows_b = rows[b]
        outv_b = outvs[b]

        def group_body(g, gcarry):
            def edge_body(j, vec):
                e = g * L + j
                acc = jnp.zeros((L,), jnp.float32)
                for c in range(D // 32):
                    sw = lax.bitcast_convert_type(
                        srows_b[e, pl.ds(c * L, L)], jnp.int32)
                    tw = lax.bitcast_convert_type(
                        drows_b[e, pl.ds(c * L, L)], jnp.int32)
                    ww = lax.bitcast_convert_type(
                        wrows_b[e, pl.ds(c * L, L)], jnp.int32)
                    s0 = lax.bitcast_convert_type(sw << 16, jnp.float32)
                    s1 = lax.bitcast_convert_type(
                        sw & jnp.int32(-65536), jnp.float32)
                    t0 = lax.bitcast_convert_type(tw << 16, jnp.float32)
                    t1 = lax.bitcast_convert_type(
                        tw & jnp.int32(-65536), jnp.float32)
                    w0 = lax.bitcast_convert_type(ww << 16, jnp.float32)
                    w1 = lax.bitcast_convert_type(
                        ww & jnp.int32(-65536), jnp.float32)
                    acc = acc + s0 * t0 * w0 + s1 * t1 * w1
                for sh in (8, 4, 2, 1):
                    acc = acc + acc.at[lane ^ sh].get(
                        mode="promise_in_bounds")
                return jnp.where(lane == j, acc, vec)

            vec = lax.fori_loop(0, L, edge_body, jnp.zeros((L,), jnp.float32))
            outv_b[pl.ds(g * L, L)] = 1.0 / (1.0 + jnp.exp(-vec))
            return gcarry

        lax.fori_loop(0, GROUPS, group_body, 0)

    npairs = half // NBUF

    def pair_body(h, jj, carry):
        for b in range(NBUF):
            j = jj * NBUF + b
            wait_rows(b)

            if h == 0:
                @pl.when(jj > 0)
                def _():
                    wait_out(b)
            else:
                wait_out(b)

            compute(b)
            gbase = tbase + (h * half + j) * SUB
            pltpu.async_copy(outvs[b], out_hbm.at[pl.ds(gbase, SUB)],
                             osems[b])

            @pl.when(jj < npairs - 1)
            def _():
                fire(j + NBUF, b)

        return carry

    for h in range(2):
        r0 = row0 + h * half
        pltpu.sync_copy(src_hbm.at[pl.ds(r0, HALF_MAX)], sidx)
        pltpu.sync_copy(dst_hbm.at[pl.ds(r0, HALF_MAX)], didx)
        pltpu.sync_copy(et_hbm.at[pl.ds(r0, HALF_MAX)], eidx)
        fire(0, 0)
        fire(1, 1)
        lax.fori_loop(0, npairs, functools.partial(pair_body, h), 0)

    wait_out(0)
    wait_out(1)


@functools.partial(jax.jit, static_argnames=("e_pad",))
def _decode(src, dst, et, z, weight, e_pad):
    mesh = plsc.VectorSubcoreMesh(core_axis_name="c", subcore_axis_name="s",
                                  num_cores=NC, num_subcores=NS)
    kern = pl.kernel(
        _sc_body,
        out_type=jax.ShapeDtypeStruct((e_pad,), jnp.float32),
        mesh=mesh,
        compiler_params=pltpu.CompilerParams(use_tc_tiling_on_sc=False),
        scratch_types=[
            pltpu.VMEM((HALF_MAX, SUB), jnp.int32),
            pltpu.VMEM((HALF_MAX, SUB), jnp.int32),
            pltpu.VMEM((HALF_MAX, SUB), jnp.int32),
            pltpu.VMEM((SUB, D // 2), jnp.float32),
            pltpu.VMEM((SUB, D // 2), jnp.float32),
            pltpu.VMEM((SUB, D // 2), jnp.float32),
            pltpu.VMEM((SUB, D // 2), jnp.float32),
            pltpu.VMEM((SUB, D // 2), jnp.float32),
            pltpu.VMEM((SUB, D // 2), jnp.float32),
            pltpu.VMEM((SUB,), jnp.float32),
            pltpu.VMEM((SUB,), jnp.float32),
            pltpu.SemaphoreType.DMA,
            pltpu.SemaphoreType.DMA,
            pltpu.SemaphoreType.DMA,
            pltpu.SemaphoreType.DMA,
        ],
    )
    return kern(src.reshape(-1, SUB), dst.reshape(-1, SUB),
                et.reshape(-1, SUB), z, weight)


def kernel(z, edge_index, edge_type, weight):
    e = edge_type.shape[0]
    e_pad = NS * (A_SUB + B_SUB) * SUB
    assert e_pad >= e
    idx_rows = NS * A_SUB + (NS - 1) * B_SUB + B_SUB // 2 + HALF_MAX
    idx_len = max(idx_rows * SUB, e_pad)
    src = edge_index[0].astype(jnp.int32)
    dst = edge_index[1].astype(jnp.int32)
    et = edge_type.astype(jnp.int32)
    if idx_len != e:
        pad = idx_len - e
        zeros = jnp.zeros((pad,), jnp.int32)
        src = jnp.concatenate([src, zeros])
        dst = jnp.concatenate([dst, zeros])
        et = jnp.concatenate([et, zeros])
    zpacked = lax.bitcast_convert_type(
        z.astype(jnp.bfloat16).reshape(z.shape[0], D // 2, 2), jnp.float32)
    wpacked = lax.bitcast_convert_type(
        weight.astype(jnp.bfloat16).reshape(weight.shape[0], D // 2, 2),
        jnp.float32)
    out = _decode(src, dst, et, zpacked, wpacked, e_pad)
    return out[:e]


# speedup vs baseline: 1.0754x; 1.0754x over previous
import functools

import jax
import jax.numpy as jnp
from jax import lax
from jax.experimental import pallas as pl
from jax.experimental.pallas import tpu as pltpu
from jax.experimental.pallas import tpu_sc as plsc

NC, NS, L = 2, 16, 16
NW = NC * NS
D = 128
SUB = 128
GROUPS = SUB // L
NBUF = 2

A_SUB = 152
B_SUB = 8
HALF_MAX = max(A_SUB, B_SUB) // 2


def _sc_body(src_hbm, dst_hbm, et_hbm, z_hbm, w_hbm, out_hbm,
             sidx, didx, eidx,
             srows0, drows0, wrows0, srows1, drows1, wrows1,
             outv0, outv1, gsem0, gsem1, osem0, osem1):
    c = lax.axis_index("c")
    s = lax.axis_index("s")
    lane = lax.iota(jnp.int32, L)
    base_sub = jnp.where(c == 0, s * A_SUB, NS * A_SUB + s * B_SUB)
    my_n = jnp.where(c == 0, A_SUB, B_SUB)
    tbase = base_sub * SUB
    row0 = base_sub

    half = my_n // 2
    rows = ((srows0, drows0, wrows0), (srows1, drows1, wrows1))
    outvs = (outv0, outv1)
    gsems = (gsem0, gsem1)
    osems = (osem0, osem1)

    def fire(j, b):
        s, d, w = rows[b]
        pltpu.async_copy(z_hbm.at[sidx.at[j]], s, gsems[b])
        pltpu.async_copy(z_hbm.at[didx.at[j]], d, gsems[b])
        pltpu.async_copy(w_hbm.at[eidx.at[j]], w, gsems[b])

    def wait_rows(b):
        s, d, w = rows[b]
        pltpu.make_async_copy(z_hbm.at[sidx.at[0]], s, gsems[b]).wait()
        pltpu.make_async_copy(z_hbm.at[didx.at[0]], d, gsems[b]).wait()
        pltpu.make_async_copy(w_hbm.at[eidx.at[0]], w, gsems[b]).wait()

    def wait_out(b):
        pltpu.make_async_copy(outvs[b], out_hbm.at[pl.ds(tbase, SUB)],
                              osems[b]).wait()

    def compute(b):
        srows_b, drows_b, wrows_b = rows[b]
        outv_b = outvs[b]

        def group_body(g, gcarry):
            def edge_body(j, vec):
                e = g * L + j
                acc = jnp.zeros((L,), jnp.float32)
                for c in range(D // 32):
                    sw = lax.bitcast_convert_type(
                        srows_b[e, pl.ds(c * L, L)], jnp.int32)
                    tw = lax.bitcast_convert_type(
                        drows_b[e, pl.ds(c * L, L)], jnp.int32)
                    ww = lax.bitcast_convert_type(
                        wrows_b[e, pl.ds(c * L, L)], jnp.int32)
                    s0 = lax.bitcast_convert_type(sw << 16, jnp.float32)
                    s1 = lax.bitcast_convert_type(
                        sw & jnp.int32(-65536), jnp.float32)
                    t0 = lax.bitcast_convert_type(tw << 16, jnp.float32)
                    t1 = lax.bitcast_convert_type(
                        tw & jnp.int32(-65536), jnp.float32)
                    w0 = lax.bitcast_convert_type(ww << 16, jnp.float32)
                    w1 = lax.bitcast_convert_type(
                        ww & jnp.int32(-65536), jnp.float32)
                    acc = acc + s0 * t0 * w0 + s1 * t1 * w1
                for sh in (8, 4, 2, 1):
                    acc = acc + acc.at[lane ^ sh].get(
                        mode="promise_in_bounds")
                return jnp.where(lane == j, acc, vec)

            vec = lax.fori_loop(0, L, edge_body, jnp.zeros((L,), jnp.float32))
            outv_b[pl.ds(g * L, L)] = 1.0 / (1.0 + jnp.exp(-vec))
            return gcarry

        lax.fori_loop(0, GROUPS, group_body, 0)

    npairs = half // NBUF

    def pair_body(h, jj, carry):
        for b in range(NBUF):
            j = jj * NBUF + b
            wait_rows(b)

            if h == 0:
                @pl.when(jj > 0)
                def _():
                    wait_out(b)
            else:
                wait_out(b)

            compute(b)
            gbase = tbase + (h * half + j) * SUB
            pltpu.async_copy(outvs[b], out_hbm.at[pl.ds(gbase, SUB)],
                             osems[b])

            @pl.when(jj < npairs - 1)
            def _():
                fire(j + NBUF, b)

        return carry

    for h in range(2):
        r0 = row0 + h * half
        pltpu.sync_copy(src_hbm.at[pl.ds(r0, HALF_MAX)], sidx)
        pltpu.sync_copy(dst_hbm.at[pl.ds(r0, HALF_MAX)], didx)
        pltpu.sync_copy(et_hbm.at[pl.ds(r0, HALF_MAX)], eidx)
        fire(0, 0)
        fire(1, 1)
        lax.fori_loop(0, npairs, functools.partial(pair_body, h), 0)

    wait_out(0)
    wait_out(1)


@functools.partial(jax.jit, static_argnames=("e_pad",))
def _decode(src, dst, et, z, weight, e_pad):
    mesh = plsc.VectorSubcoreMesh(core_axis_name="c", subcore_axis_name="s",
                                  num_cores=NC, num_subcores=NS)
    kern = pl.kernel(
        _sc_body,
        out_type=jax.ShapeDtypeStruct((e_pad,), jnp.float32),
        mesh=mesh,
        compiler_params=pltpu.CompilerParams(use_tc_tiling_on_sc=False),
        scratch_types=[
            pltpu.VMEM((HALF_MAX, SUB), jnp.int32),
            pltpu.VMEM((HALF_MAX, SUB), jnp.int32),
            pltpu.VMEM((HALF_MAX, SUB), jnp.int32),
            pltpu.VMEM((SUB, D // 2), jnp.float32),
            pltpu.VMEM((SUB, D // 2), jnp.float32),
            pltpu.VMEM((SUB, D // 2), jnp.float32),
            pltpu.VMEM((SUB, D // 2), jnp.float32),
            pltpu.VMEM((SUB, D // 2), jnp.float32),
            pltpu.VMEM((SUB, D // 2), jnp.float32),
            pltpu.VMEM((SUB,), jnp.float32),
            pltpu.VMEM((SUB,), jnp.float32),
            pltpu.SemaphoreType.DMA,
            pltpu.SemaphoreType.DMA,
            pltpu.SemaphoreType.DMA,
            pltpu.SemaphoreType.DMA,
        ],
    )
    return kern(src.reshape(-1, SUB), dst.reshape(-1, SUB),
                et.reshape(-1, SUB), z, weight)


def kernel(z, edge_index, edge_type, weight):
    e = edge_type.shape[0]
    e_pad = NS * (A_SUB + B_SUB) * SUB
    assert e_pad >= e
    idx_rows = NS * A_SUB + (NS - 1) * B_SUB + B_SUB // 2 + HALF_MAX
    idx_len = max(idx_rows * SUB, e_pad)
    src = edge_index[0].astype(jnp.int32)
    dst = edge_index[1].astype(jnp.int32)
    et = edge_type.astype(jnp.int32)
    if idx_len != e:
        pad = idx_len - e
        zeros = jnp.zeros((pad,), jnp.int32)
        src = jnp.concatenate([src, zeros])
        dst = jnp.concatenate([dst, zeros])
        et = jnp.concatenate([et, zeros])
    zpacked = lax.bitcast_convert_type(
        z.astype(jnp.bfloat16).reshape(z.shape[0], D // 2, 2), jnp.float32)
    wpacked = lax.bitcast_convert_type(
        weight.astype(jnp.bfloat16).reshape(weight.shape[0], D // 2, 2),
        jnp.float32)
    out = _decode(src, dst, et, zpacked, wpacked, e_pad)
    return out[:e]
